# baseline (speedup 1.0000x reference)
import functools
import math

import jax
import jax.numpy as jnp
from jax import lax
from jax.experimental import pallas as pl
from jax.experimental.pallas import tpu as pltpu
from jax.experimental.pallas import tpu_sc as plsc

D_MODEL = 128
VOCAB = 1000
SCALE = math.sqrt(float(D_MODEL))

BATCH = 1024
SEQ = 50
ROWS = BATCH * SEQ

S_BLK = 1
L_BLK = 1024
S_STEPS = SEQ // S_BLK
L_STEPS = BATCH // L_BLK

NUM_CORES = 2
NUM_SUBCORES = 16
NW = NUM_CORES * NUM_SUBCORES
B_PER_W = ROWS // NW
CHUNK = 80
N_CHUNK = B_PER_W // CHUNK


def _tc_argmax_body(x_ref, t_ref, idx_ref, st_ref):
    xb = x_ref[...]
    m = jnp.max(xb, axis=1, keepdims=True)
    ii = lax.broadcasted_iota(jnp.int32, xb.shape, 1)
    cand = jnp.where(xb == m, ii, VOCAB)
    idx_ref[...] = jnp.min(cand, axis=1)[:, None, :]

    @pl.when((pl.program_id(0) == 0) & (pl.program_id(1) == 0))
    def _():
        st_ref[...] = t_ref[...] * SCALE


_tc_argmax = pl.pallas_call(
    _tc_argmax_body,
    grid=(S_STEPS, L_STEPS),
    in_specs=[
        pl.BlockSpec((S_BLK, VOCAB, L_BLK), lambda i, j: (i, 0, j)),
        pl.BlockSpec((VOCAB, D_MODEL), lambda i, j: (0, 0)),
    ],
    out_specs=[
        pl.BlockSpec((S_BLK, 1, L_BLK), lambda i, j: (i, 0, j)),
        pl.BlockSpec((VOCAB, D_MODEL), lambda i, j: (0, 0)),
    ],
    out_shape=[
        jax.ShapeDtypeStruct((SEQ, 1, BATCH), jnp.int32),
        jax.ShapeDtypeStruct((VOCAB, D_MODEL), jnp.float32),
    ],
    compiler_params=pltpu.CompilerParams(
        dimension_semantics=("arbitrary", "arbitrary"),
    ),
)


@functools.lru_cache(maxsize=1)
def _build_sc_gather():
    mesh = plsc.VectorSubcoreMesh(
        core_axis_name="c",
        subcore_axis_name="s",
        num_cores=NUM_CORES,
        num_subcores=NUM_SUBCORES,
    )

    @functools.partial(
        pl.kernel,
        out_type=jax.ShapeDtypeStruct((ROWS, D_MODEL), jnp.float32),
        mesh=mesh,
        scratch_types=[
            pltpu.VMEM((CHUNK,), jnp.int32),
            pltpu.VMEM((CHUNK, D_MODEL), jnp.float32),
            pltpu.SemaphoreType.DMA,
        ],
    )
    def sc_gather(table_hbm, idx_hbm, out_hbm, idx_v, rows_v, sem):
        wid = lax.axis_index("s") * NUM_CORES + lax.axis_index("c")
        base = wid * B_PER_W

        def body(c, carry):
            off = base + c * CHUNK
            pltpu.sync_copy(idx_hbm.at[pl.ds(off, CHUNK)], idx_v)
            pltpu.async_copy(table_hbm.at[idx_v], rows_v, sem).wait()
            pltpu.sync_copy(rows_v, out_hbm.at[pl.ds(off, CHUNK)])
            return carry

        lax.fori_loop(0, N_CHUNK, body, 0)

    return sc_gather


def kernel(x, table):
    b, s, v = x.shape
    xt = jnp.transpose(x, (1, 2, 0))
    idx, scaled_table = _tc_argmax(xt, table)
    idx_flat = idx.reshape(s * b)
    out = _build_sc_gather()(scaled_table, idx_flat)
    out3 = out.reshape(s, b, D_MODEL)
    return jnp.transpose(out3, (1, 0, 2))


# speedup vs baseline: 1.1037x; 1.1037x over previous
import functools
import math

import jax
import jax.numpy as jnp
from jax import lax
from jax.experimental import pallas as pl
from jax.experimental.pallas import tpu as pltpu
from jax.experimental.pallas import tpu_sc as plsc

D_MODEL = 128
VOCAB = 1000
SCALE = math.sqrt(float(D_MODEL))

BATCH = 1024
SEQ = 50
ROWS = BATCH * SEQ

S_BLK = 5
L_BLK = 1024
S_STEPS = SEQ // S_BLK
L_STEPS = BATCH // L_BLK

NUM_CORES = 2
NUM_SUBCORES = 16
NW = NUM_CORES * NUM_SUBCORES
B_PER_W = ROWS // NW
CHUNK = 80
N_CHUNK = B_PER_W // CHUNK


def _tc_argmax_body(x_ref, t_ref, idx_ref, st_ref):
    xb = x_ref[...]
    m = jnp.max(xb, axis=1, keepdims=True)
    ii = lax.broadcasted_iota(jnp.int32, xb.shape, 1)
    cand = jnp.where(xb == m, ii, VOCAB)
    idx_ref[...] = jnp.min(cand, axis=1)[:, None, :]

    @pl.when((pl.program_id(0) == 0) & (pl.program_id(1) == 0))
    def _():
        st_ref[...] = t_ref[...] * SCALE


_tc_argmax = pl.pallas_call(
    _tc_argmax_body,
    grid=(S_STEPS, L_STEPS),
    in_specs=[
        pl.BlockSpec((S_BLK, VOCAB, L_BLK), lambda i, j: (i, 0, j)),
        pl.BlockSpec((VOCAB, D_MODEL), lambda i, j: (0, 0)),
    ],
    out_specs=[
        pl.BlockSpec((S_BLK, 1, L_BLK), lambda i, j: (i, 0, j)),
        pl.BlockSpec((VOCAB, D_MODEL), lambda i, j: (0, 0)),
    ],
    out_shape=[
        jax.ShapeDtypeStruct((SEQ, 1, BATCH), jnp.int32),
        jax.ShapeDtypeStruct((VOCAB, D_MODEL), jnp.float32),
    ],
    compiler_params=pltpu.CompilerParams(
        dimension_semantics=("arbitrary", "arbitrary"),
    ),
)


@functools.lru_cache(maxsize=1)
def _build_sc_gather():
    mesh = plsc.VectorSubcoreMesh(
        core_axis_name="c",
        subcore_axis_name="s",
        num_cores=NUM_CORES,
        num_subcores=NUM_SUBCORES,
    )

    @functools.partial(
        pl.kernel,
        out_type=jax.ShapeDtypeStruct((ROWS, D_MODEL), jnp.float32),
        mesh=mesh,
        scratch_types=[
            pltpu.VMEM((CHUNK,), jnp.int32),
            pltpu.VMEM((CHUNK, D_MODEL), jnp.float32),
            pltpu.SemaphoreType.DMA,
        ],
    )
    def sc_gather(table_hbm, idx_hbm, out_hbm, idx_v, rows_v, sem):
        wid = lax.axis_index("s") * NUM_CORES + lax.axis_index("c")
        base = wid * B_PER_W

        def body(c, carry):
            off = base + c * CHUNK
            pltpu.sync_copy(idx_hbm.at[pl.ds(off, CHUNK)], idx_v)
            pltpu.async_copy(table_hbm.at[idx_v], rows_v, sem).wait()
            pltpu.sync_copy(rows_v, out_hbm.at[pl.ds(off, CHUNK)])
            return carry

        lax.fori_loop(0, N_CHUNK, body, 0)

    return sc_gather


def kernel(x, table):
    b, s, v = x.shape
    xt = jnp.transpose(x, (1, 2, 0))
    idx, scaled_table = _tc_argmax(xt, table)
    idx_flat = idx.reshape(s * b)
    out = _build_sc_gather()(scaled_table, idx_flat)
    out3 = out.reshape(s, b, D_MODEL)
    return jnp.transpose(out3, (1, 0, 2))


# speedup vs baseline: 1.2638x; 1.1451x over previous
import functools
import math

import jax
import jax.numpy as jnp
from jax import lax
from jax.experimental import pallas as pl
from jax.experimental.pallas import tpu as pltpu
from jax.experimental.pallas import tpu_sc as plsc

D_MODEL = 128
VOCAB = 1000
SCALE = math.sqrt(float(D_MODEL))

BATCH = 1024
SEQ = 50
ROWS = BATCH * SEQ

S_BLK = 5
L_BLK = 1024
S_STEPS = SEQ // S_BLK
L_STEPS = BATCH // L_BLK

NUM_CORES = 2
NUM_SUBCORES = 16
NW = NUM_CORES * NUM_SUBCORES
B_PER_W = ROWS // NW
CHUNK = 80
N_CHUNK = B_PER_W // CHUNK


def _tc_argmax_body(x_ref, t_ref, idx_ref, st_ref):
    xb = x_ref[...]
    m = jnp.max(xb, axis=1, keepdims=True)
    ii = lax.broadcasted_iota(jnp.int32, xb.shape, 1)
    cand = jnp.where(xb == m, ii, VOCAB)
    idx_ref[...] = jnp.min(cand, axis=1)[:, None, :]

    @pl.when((pl.program_id(0) == 0) & (pl.program_id(1) == 0))
    def _():
        st_ref[...] = t_ref[...] * SCALE


_tc_argmax = pl.pallas_call(
    _tc_argmax_body,
    grid=(S_STEPS, L_STEPS),
    in_specs=[
        pl.BlockSpec((S_BLK, VOCAB, L_BLK), lambda i, j: (i, 0, j)),
        pl.BlockSpec((VOCAB, D_MODEL), lambda i, j: (0, 0)),
    ],
    out_specs=[
        pl.BlockSpec((S_BLK, 1, L_BLK), lambda i, j: (i, 0, j)),
        pl.BlockSpec((VOCAB, D_MODEL), lambda i, j: (0, 0)),
    ],
    out_shape=[
        jax.ShapeDtypeStruct((SEQ, 1, BATCH), jnp.int32),
        jax.ShapeDtypeStruct((VOCAB, D_MODEL), jnp.float32),
    ],
    compiler_params=pltpu.CompilerParams(
        dimension_semantics=("arbitrary", "arbitrary"),
    ),
)


@functools.lru_cache(maxsize=1)
def _build_sc_gather():
    mesh = plsc.VectorSubcoreMesh(
        core_axis_name="c",
        subcore_axis_name="s",
        num_cores=NUM_CORES,
        num_subcores=NUM_SUBCORES,
    )

    NB = 5

    @functools.partial(
        pl.kernel,
        out_type=jax.ShapeDtypeStruct((ROWS, D_MODEL), jnp.float32),
        mesh=mesh,
        scratch_types=[
            pltpu.VMEM((B_PER_W,), jnp.int32),
            pltpu.VMEM((NB, CHUNK, D_MODEL), jnp.float32),
            pltpu.SemaphoreType.DMA,
            pltpu.SemaphoreType.DMA,
        ],
    )
    def sc_gather(table_hbm, idx_hbm, out_hbm, idx_v, rows, gsem, ssem):
        wid = lax.axis_index("s") * NUM_CORES + lax.axis_index("c")
        base = wid * B_PER_W
        pltpu.sync_copy(idx_hbm.at[pl.ds(base, B_PER_W)], idx_v)
        for b in range(NB):
            pltpu.async_copy(
                table_hbm.at[idx_v.at[pl.ds(b * CHUNK, CHUNK)]], rows.at[b], gsem
            )
        for c in range(N_CHUNK):
            b = c % NB
            out_slice = out_hbm.at[pl.ds(base + c * CHUNK, CHUNK)]
            pltpu.make_async_copy(
                table_hbm.at[idx_v.at[pl.ds(c * CHUNK, CHUNK)]], rows.at[b], gsem
            ).wait()
            pltpu.async_copy(rows.at[b], out_slice, ssem)
            nxt = c + NB
            if nxt < N_CHUNK:
                pltpu.make_async_copy(rows.at[b], out_slice, ssem).wait()
                pltpu.async_copy(
                    table_hbm.at[idx_v.at[pl.ds(nxt * CHUNK, CHUNK)]],
                    rows.at[b],
                    gsem,
                )
        for c in range(N_CHUNK - NB, N_CHUNK):
            b = c % NB
            pltpu.make_async_copy(
                rows.at[b], out_hbm.at[pl.ds(base + c * CHUNK, CHUNK)], ssem
            ).wait()

    return sc_gather


def kernel(x, table):
    b, s, v = x.shape
    xt = jnp.transpose(x, (1, 2, 0))
    idx, scaled_table = _tc_argmax(xt, table)
    idx_flat = idx.reshape(s * b)
    out = _build_sc_gather()(scaled_table, idx_flat)
    out3 = out.reshape(s, b, D_MODEL)
    return jnp.transpose(out3, (1, 0, 2))


# speedup vs baseline: 1.4938x; 1.1820x over previous
import functools
import math

import jax
import jax.numpy as jnp
from jax import lax
from jax.experimental import pallas as pl
from jax.experimental.pallas import tpu as pltpu
from jax.experimental.pallas import tpu_sc as plsc

D_MODEL = 128
VOCAB = 1000
SCALE = math.sqrt(float(D_MODEL))

BATCH = 1024
SEQ = 50
ROWS = BATCH * SEQ

S_BLK = 5
L_BLK = 1024
S_STEPS = SEQ // S_BLK
L_STEPS = BATCH // L_BLK

NUM_CORES = 2
NUM_SUBCORES = 16
NW = NUM_CORES * NUM_SUBCORES
B_PER_W = ROWS // NW
CHUNK = 80
N_CHUNK = B_PER_W // CHUNK


def _tc_argmax_body(x_ref, t_ref, idx_ref, st_ref):
    xb = x_ref[...]
    m = jnp.max(xb, axis=1, keepdims=True)
    ii = lax.broadcasted_iota(jnp.int32, xb.shape, 1)
    cand = jnp.where(xb == m, ii, VOCAB)
    idx_ref[...] = jnp.min(cand, axis=1)[:, None, :]

    @pl.when((pl.program_id(0) == 0) & (pl.program_id(1) == 0))
    def _():
        st_ref[...] = t_ref[...] * SCALE


_tc_argmax = pl.pallas_call(
    _tc_argmax_body,
    grid=(S_STEPS, L_STEPS),
    in_specs=[
        pl.BlockSpec((S_BLK, VOCAB, L_BLK), lambda i, j: (i, 0, j)),
        pl.BlockSpec((VOCAB, D_MODEL), lambda i, j: (0, 0)),
    ],
    out_specs=[
        pl.BlockSpec((S_BLK, 1, L_BLK), lambda i, j: (i, 0, j)),
        pl.BlockSpec((VOCAB, D_MODEL), lambda i, j: (0, 0)),
    ],
    out_shape=[
        jax.ShapeDtypeStruct((SEQ, 1, BATCH), jnp.int32),
        jax.ShapeDtypeStruct((VOCAB, D_MODEL), jnp.float32),
    ],
    compiler_params=pltpu.CompilerParams(
        dimension_semantics=("arbitrary", "arbitrary"),
    ),
)


@functools.lru_cache(maxsize=1)
def _build_sc_gather():
    mesh = plsc.VectorSubcoreMesh(
        core_axis_name="c",
        subcore_axis_name="s",
        num_cores=NUM_CORES,
        num_subcores=NUM_SUBCORES,
    )

    NB = 5

    @functools.partial(
        pl.kernel,
        out_type=jax.ShapeDtypeStruct((ROWS, D_MODEL), jnp.float32),
        mesh=mesh,
        scratch_types=[
            pltpu.VMEM((B_PER_W,), jnp.int32),
            pltpu.VMEM((NB, CHUNK, D_MODEL), jnp.float32),
            pltpu.VMEM_SHARED((VOCAB, D_MODEL), jnp.float32),
            pltpu.SemaphoreType.DMA,
            pltpu.SemaphoreType.DMA,
        ],
    )
    def sc_gather(table_hbm, idx_hbm, out_hbm, idx_v, rows, table_sh, gsem, ssem):
        wid = lax.axis_index("s") * NUM_CORES + lax.axis_index("c")
        base = wid * B_PER_W
        @pl.when(lax.axis_index("s") == 0)
        def _():
            pltpu.sync_copy(table_hbm, table_sh)

        pltpu.sync_copy(idx_hbm.at[pl.ds(base, B_PER_W)], idx_v)
        plsc.subcore_barrier()
        for b in range(NB):
            pltpu.async_copy(
                table_sh.at[idx_v.at[pl.ds(b * CHUNK, CHUNK)]], rows.at[b], gsem
            )
        for c in range(N_CHUNK):
            b = c % NB
            out_slice = out_hbm.at[pl.ds(base + c * CHUNK, CHUNK)]
            pltpu.make_async_copy(
                table_sh.at[idx_v.at[pl.ds(c * CHUNK, CHUNK)]], rows.at[b], gsem
            ).wait()
            pltpu.async_copy(rows.at[b], out_slice, ssem)
            nxt = c + NB
            if nxt < N_CHUNK:
                pltpu.make_async_copy(rows.at[b], out_slice, ssem).wait()
                pltpu.async_copy(
                    table_sh.at[idx_v.at[pl.ds(nxt * CHUNK, CHUNK)]],
                    rows.at[b],
                    gsem,
                )
        for c in range(N_CHUNK - NB, N_CHUNK):
            b = c % NB
            pltpu.make_async_copy(
                rows.at[b], out_hbm.at[pl.ds(base + c * CHUNK, CHUNK)], ssem
            ).wait()

    return sc_gather


def kernel(x, table):
    b, s, v = x.shape
    xt = jnp.transpose(x, (1, 2, 0))
    idx, scaled_table = _tc_argmax(xt, table)
    idx_flat = idx.reshape(s * b)
    out = _build_sc_gather()(scaled_table, idx_flat)
    out3 = out.reshape(s, b, D_MODEL)
    return jnp.transpose(out3, (1, 0, 2))
